# Initial kernel scaffold; baseline (speedup 1.0000x reference)
import jax
import jax.numpy as jnp
from jax.experimental import pallas as pl


def kernel(x, edge_index, batch, W1, b1, g1, be1, W2, b2, g2, be2, W3, b3, g3, be3, fW1, fb1, fW2, fb2, fW3, fb3):
    raise NotImplementedError("write your pallas kernel here")


# speedup vs baseline: 12.7092x; 12.7092x over previous
import functools

import jax
import jax.numpy as jnp
from jax import lax
from jax.experimental import pallas as pl
from jax.experimental.pallas import tpu as pltpu
from jax.experimental.pallas import tpu_sc as plsc

N = 50000
E = 800000
F = 10
H = 128
L = 256
G = 512
EPS = 1e-5

NPAD = 50688
BLK = 512
NBLK = NPAD // BLK
CHUNK = 8448
NCHUNK = 6
STRIPE = CHUNK // 16
EPAD = 802816
EPT = EPAD // 32
ESB = EPT // 4
CAP = 4608
NB = CAP // 128
NLIST = NCHUNK * 32 * CAP

f32 = jnp.float32
i32 = jnp.int32


def _mesh():
    return plsc.VectorSubcoreMesh(core_axis_name="c", subcore_axis_name="s")


def _sc_filter(src_p, dst_p):
    @functools.partial(
        pl.kernel,
        out_type=(
            jax.ShapeDtypeStruct((NLIST,), i32),
            jax.ShapeDtypeStruct((NLIST,), i32),
        ),
        mesh=_mesh(),
        compiler_params=pltpu.CompilerParams(needs_layout_passes=False),
        scratch_types=[
            pltpu.VMEM((NCHUNK * CAP,), i32),
            pltpu.VMEM((NCHUNK * CAP,), i32),
            pltpu.VMEM((ESB,), i32),
            pltpu.VMEM((ESB,), i32),
        ],
    )
    def k(src_hbm, dst_hbm, lsrc_hbm, ldst_hbm, bs, bd, sblk, dblk):
        cid = lax.axis_index("c")
        sid = lax.axis_index("s")
        tid = cid * 16 + sid
        iv = lax.iota(i32, 16)

        def pf(kk, _):
            sv = N + lax.rem(iv + kk, 176)
            dv = lax.rem((iv + kk * 29) * 97, CHUNK)
            for c in range(NCHUNK):
                bs[pl.ds(c * CAP + kk * 16, 16)] = sv
                bd[pl.ds(c * CAP + kk * 16, 16)] = dv
            return 0

        lax.fori_loop(0, CAP // 16, pf, 0)

        base = pl.multiple_of(tid * EPT, 8)
        cnts = (jnp.zeros((), i32),) * NCHUNK
        for sb in range(4):
            pltpu.sync_copy(src_hbm.at[pl.ds(base + sb * ESB, ESB)], sblk)
            pltpu.sync_copy(dst_hbm.at[pl.ds(base + sb * ESB, ESB)], dblk)

            def fb(kk, cs):
                sl = pl.ds(kk * 16, 16)
                d = dblk[sl]
                s = sblk[sl]
                ch = lax.div(d, CHUNK)
                off = d - ch * CHUNK
                out = []
                for c in range(NCHUNK):
                    msk = ch == c
                    n = jnp.sum(msk.astype(i32))
                    cc = jnp.minimum(cs[c], CAP - 16)
                    plsc.store_compressed(bs.at[pl.ds(c * CAP + cc, 16)], s,
                                          mask=msk)
                    plsc.store_compressed(bd.at[pl.ds(c * CAP + cc, 16)], off,
                                          mask=msk)
                    out.append(cc + n)
                return tuple(out)

            cnts = lax.fori_loop(0, ESB // 16, fb, cnts)

        for c in range(NCHUNK):
            lb = pl.multiple_of((c * 32 + tid) * CAP, 8)
            pltpu.sync_copy(bs.at[pl.ds(c * CAP, CAP)],
                            lsrc_hbm.at[pl.ds(lb, CAP)])
            pltpu.sync_copy(bd.at[pl.ds(c * CAP, CAP)],
                            ldst_hbm.at[pl.ds(lb, CAP)])

    return k(src_p, dst_p)


def _sc_agg(table, lsrc, ldst, w):
    @functools.partial(
        pl.kernel,
        out_type=jax.ShapeDtypeStruct((NPAD, w), f32),
        mesh=_mesh(),
        compiler_params=pltpu.CompilerParams(needs_layout_passes=False,
                                             use_tc_tiling_on_sc=False),
        scratch_types=[
            pltpu.VMEM((CAP,), i32),
            pltpu.VMEM((CAP,), i32),
            pltpu.VMEM((128,), i32),
            pltpu.VMEM((128,), i32),
            pltpu.VMEM((128, w), f32),
            pltpu.VMEM_SHARED((CHUNK, w), f32),
            pltpu.SemaphoreType.DMA,
        ],
    )
    def k(tab, lsrc_h, ldst_h, m_h, sbuf, dbuf, sidx, didx, rows, acc, sem):
        cid = lax.axis_index("c")
        sid = lax.axis_index("s")
        z16 = jnp.zeros((16,), f32)
        sb0 = pl.multiple_of(sid * STRIPE, 16)
        for ci in range(3):
            c = 3 * cid + ci

            def zr(i, _):
                r = i // (w // 16)
                col = lax.rem(i, w // 16) * 16
                rows[r, pl.ds(col, 16)] = z16
                return 0

            lax.fori_loop(0, 128 * (w // 16), zr, 0)
            for q in range(4):
                pltpu.sync_copy(rows, acc.at[pl.ds(sb0 + q * 128, 128)])
            pltpu.sync_copy(rows.at[pl.ds(0, 16)], acc.at[pl.ds(sb0 + 512, 16)])
            plsc.subcore_barrier()

            for jj in range(2):
                j = 2 * sid + jj
                lb = pl.multiple_of((c * 32 + j) * CAP, 8)
                pltpu.sync_copy(lsrc_h.at[pl.ds(lb, CAP)], sbuf)
                pltpu.sync_copy(ldst_h.at[pl.ds(lb, CAP)], dbuf)

                def bat(b, _):
                    def cp(kk, _2):
                        sl = pl.ds(kk * 16, 16)
                        sidx[sl] = sbuf[pl.ds(b * 128 + kk * 16, 16)]
                        didx[sl] = dbuf[pl.ds(b * 128 + kk * 16, 16)]
                        return 0

                    lax.fori_loop(0, 8, cp, 0)
                    pltpu.async_copy(tab.at[sidx], rows, sem).wait()
                    pltpu.sync_copy(rows, acc.at[didx], add=True)
                    return 0

                lax.fori_loop(0, NB, bat, 0)
            plsc.subcore_barrier()
            mb = c * CHUNK + sb0
            pltpu.sync_copy(acc.at[pl.ds(sb0, STRIPE)], m_h.at[pl.ds(mb, STRIPE)])
            plsc.subcore_barrier()

    return k(table, lsrc, ldst)


def _rowid(i):
    return lax.broadcasted_iota(i32, (BLK, 1), 0) + i * BLK


def _tc_prep(m1, xpad):
    def body(dp_ref, x_ref, dinv_ref, gx_ref):
        i = pl.program_id(0)
        deg = dp_ref[:, 0:1] + 1.0
        dinv = jnp.where(_rowid(i) < N,
                         lax.rsqrt(jnp.maximum(deg, 1.0)), 0.0)
        dinv_ref[...] = dinv
        gx_ref[...] = x_ref[...] * dinv

    return pl.pallas_call(
        body,
        grid=(NBLK,),
        in_specs=[
            pl.BlockSpec((BLK, 16), lambda i: (i, 0)),
            pl.BlockSpec((BLK, 16), lambda i: (i, 0)),
        ],
        out_specs=[
            pl.BlockSpec((BLK, 1), lambda i: (i, 0)),
            pl.BlockSpec((BLK, 16), lambda i: (i, 0)),
        ],
        out_shape=[
            jax.ShapeDtypeStruct((NPAD, 1), f32),
            jax.ShapeDtypeStruct((NPAD, 16), f32),
        ],
    )(m1, xpad)


def _stats_update(i, y, scr, st_ref):
    ym = jnp.where(_rowid(i) < N, y, 0.0)

    @pl.when(i == 0)
    def _():
        scr[...] = jnp.zeros_like(scr)

    scr[0:1, :] += jnp.sum(ym, axis=0, keepdims=True)
    scr[1:2, :] += jnp.sum(ym * ym, axis=0, keepdims=True)
    st_ref[...] = scr[...]


def _tc_post1(mx, gx, dinv, w1p, b1r):
    def body(m_ref, g_ref, dv_ref, w_ref, b_ref, y_ref, st_ref, scr):
        i = pl.program_id(0)
        t = dv_ref[...] * (m_ref[...] + g_ref[...])
        y = jnp.dot(t, w_ref[...], preferred_element_type=f32) + b_ref[...]
        y_ref[...] = y
        _stats_update(i, y, scr, st_ref)

    return pl.pallas_call(
        body,
        grid=(NBLK,),
        in_specs=[
            pl.BlockSpec((BLK, 16), lambda i: (i, 0)),
            pl.BlockSpec((BLK, 16), lambda i: (i, 0)),
            pl.BlockSpec((BLK, 1), lambda i: (i, 0)),
            pl.BlockSpec((16, H), lambda i: (0, 0)),
            pl.BlockSpec((1, H), lambda i: (0, 0)),
        ],
        out_specs=[
            pl.BlockSpec((BLK, H), lambda i: (i, 0)),
            pl.BlockSpec((2, H), lambda i: (0, 0)),
        ],
        out_shape=[
            jax.ShapeDtypeStruct((NPAD, H), f32),
            jax.ShapeDtypeStruct((2, H), f32),
        ],
        scratch_shapes=[pltpu.VMEM((2, H), f32)],
    )(mx, gx, dinv, w1p, b1r)


def _tc_post23(m, g, dinv, br):
    def body(m_ref, g_ref, dv_ref, b_ref, y_ref, st_ref, scr):
        i = pl.program_id(0)
        y = dv_ref[...] * (m_ref[...] + g_ref[...]) + b_ref[...]
        y_ref[...] = y
        _stats_update(i, y, scr, st_ref)

    return pl.pallas_call(
        body,
        grid=(NBLK,),
        in_specs=[
            pl.BlockSpec((BLK, H), lambda i: (i, 0)),
            pl.BlockSpec((BLK, H), lambda i: (i, 0)),
            pl.BlockSpec((BLK, 1), lambda i: (i, 0)),
            pl.BlockSpec((1, H), lambda i: (0, 0)),
        ],
        out_specs=[
            pl.BlockSpec((BLK, H), lambda i: (i, 0)),
            pl.BlockSpec((2, H), lambda i: (0, 0)),
        ],
        out_shape=[
            jax.ShapeDtypeStruct((NPAD, H), f32),
            jax.ShapeDtypeStruct((2, H), f32),
        ],
        scratch_shapes=[pltpu.VMEM((2, H), f32)],
    )(m, g, dinv, br)


def _bn_relu(y, st_ref, ga_ref, be_ref):
    mu = st_ref[0:1, :] * (1.0 / N)
    var = st_ref[1:2, :] * (1.0 / N) - mu * mu
    rstd = lax.rsqrt(var + EPS)
    return jnp.maximum(ga_ref[...] * (y - mu) * rstd + be_ref[...], 0.0)


def _tc_mid(y, st, gar, ber, dinv, wn):
    def body(y_ref, st_ref, ga_ref, be_ref, dv_ref, w_ref, gn_ref):
        h = _bn_relu(y_ref[...], st_ref, ga_ref, be_ref)
        gn_ref[...] = dv_ref[...] * jnp.dot(h, w_ref[...],
                                            preferred_element_type=f32)

    return pl.pallas_call(
        body,
        grid=(NBLK,),
        in_specs=[
            pl.BlockSpec((BLK, H), lambda i: (i, 0)),
            pl.BlockSpec((2, H), lambda i: (0, 0)),
            pl.BlockSpec((1, H), lambda i: (0, 0)),
            pl.BlockSpec((1, H), lambda i: (0, 0)),
            pl.BlockSpec((BLK, 1), lambda i: (i, 0)),
            pl.BlockSpec((H, H), lambda i: (0, 0)),
        ],
        out_specs=pl.BlockSpec((BLK, H), lambda i: (i, 0)),
        out_shape=jax.ShapeDtypeStruct((NPAD, H), f32),
    )(y, st, gar, ber, dinv, wn)


def _tc_final(y3, s3, g3r, be3r, bcol, fw1, fb1r, fw2, fb2r, fw3p, fb3p):
    def body(y_ref, st_ref, ga_ref, be_ref, b_ref, w1_ref, b1_ref, w2_ref,
             b2_ref, w3_ref, b3_ref, out_ref, pool, cnt):
        i = pl.program_id(0)
        h = _bn_relu(y_ref[...], st_ref, ga_ref, be_ref)
        oh = (b_ref[...] == lax.broadcasted_iota(i32, (1, G), 1)).astype(f32)

        @pl.when(i == 0)
        def _():
            pool[...] = jnp.zeros_like(pool)
            cnt[...] = jnp.zeros_like(cnt)

        dn = (((0,), (0,)), ((), ()))
        pool[...] += lax.dot_general(oh, h, dn, preferred_element_type=f32)
        cnt[...] += lax.dot_general(oh, jnp.ones((BLK, 1), f32), dn,
                                    preferred_element_type=f32)

        @pl.when(i == NBLK - 1)
        def _():
            pm = pool[...] / jnp.maximum(cnt[...], 1.0)
            z = jnp.maximum(
                jnp.dot(pm, w1_ref[...], preferred_element_type=f32)
                + b1_ref[...], 0.0)
            z = jnp.maximum(
                jnp.dot(z, w2_ref[...], preferred_element_type=f32)
                + b2_ref[...], 0.0)
            out_ref[...] = (jnp.dot(z, w3_ref[...], preferred_element_type=f32)
                            + b3_ref[...])

    return pl.pallas_call(
        body,
        grid=(NBLK,),
        in_specs=[
            pl.BlockSpec((BLK, H), lambda i: (i, 0)),
            pl.BlockSpec((2, H), lambda i: (0, 0)),
            pl.BlockSpec((1, H), lambda i: (0, 0)),
            pl.BlockSpec((1, H), lambda i: (0, 0)),
            pl.BlockSpec((BLK, 1), lambda i: (i, 0)),
            pl.BlockSpec((H, L), lambda i: (0, 0)),
            pl.BlockSpec((1, L), lambda i: (0, 0)),
            pl.BlockSpec((L, L), lambda i: (0, 0)),
            pl.BlockSpec((1, L), lambda i: (0, 0)),
            pl.BlockSpec((L, H), lambda i: (0, 0)),
            pl.BlockSpec((1, H), lambda i: (0, 0)),
        ],
        out_specs=pl.BlockSpec((G, H), lambda i: (0, 0)),
        out_shape=jax.ShapeDtypeStruct((G, H), f32),
        scratch_shapes=[pltpu.VMEM((G, H), f32), pltpu.VMEM((G, 1), f32)],
    )(y3, s3, g3r, be3r, bcol, fw1, fb1r, fw2, fb2r, fw3p, fb3p)


def kernel(x, edge_index, batch, W1, b1, g1, be1, W2, b2, g2, be2,
           W3, b3, g3, be3, fW1, fb1, fW2, fb2, fW3, fb3):
    extra = EPAD - E
    ar = jnp.arange(extra, dtype=i32)
    src_p = jnp.concatenate([edge_index[0], ar % 512])
    dst_p = jnp.concatenate([edge_index[1], N + ar % 176])
    xpad = jnp.pad(x, ((0, NPAD - N), (0, 16 - F)))
    bcol = jnp.pad(batch, (0, NPAD - N), constant_values=G).reshape(NPAD, 1)
    w1p = jnp.pad(W1, ((0, 16 - F), (0, 0)))
    fw3p = jnp.pad(fW3, ((0, 0), (0, H - 1)))
    fb3p = jnp.pad(fb3, (0, H - 1)).reshape(1, H)

    lsrc, ldst = _sc_filter(src_p, dst_p)
    ones_tbl = jnp.zeros((NPAD, 16), f32).at[:N, 0].set(1.0)
    m1 = _sc_agg(ones_tbl, lsrc, ldst, 16)
    dinv, gx = _tc_prep(m1, xpad)

    mx = _sc_agg(gx, lsrc, ldst, 16)
    y1, s1 = _tc_post1(mx, gx, dinv, w1p, b1.reshape(1, H))
    g2_ = _tc_mid(y1, s1, g1.reshape(1, H), be1.reshape(1, H), dinv, W2)

    m2 = _sc_agg(g2_, lsrc, ldst, H)
    y2, s2 = _tc_post23(m2, g2_, dinv, b2.reshape(1, H))
    g3_ = _tc_mid(y2, s2, g2.reshape(1, H), be2.reshape(1, H), dinv, W3)

    m3 = _sc_agg(g3_, lsrc, ldst, H)
    y3, s3 = _tc_post23(m3, g3_, dinv, b3.reshape(1, H))

    out2d = _tc_final(y3, s3, g3.reshape(1, H), be3.reshape(1, H), bcol,
                      fW1, fb1.reshape(1, L), fW2, fb2.reshape(1, L),
                      fw3p, fb3p)
    return out2d[:, 0]


# speedup vs baseline: 15.0893x; 1.1873x over previous
import functools

import jax
import jax.numpy as jnp
from jax import lax
from jax.experimental import pallas as pl
from jax.experimental.pallas import tpu as pltpu
from jax.experimental.pallas import tpu_sc as plsc

N = 50000
E = 800000
F = 10
H = 128
L = 256
G = 512
EPS = 1e-5

NPAD = 50688
BLK = 512
NBLK = NPAD // BLK
CHUNK = 8448
NCHUNK = 6
STRIPE = CHUNK // 16
EPAD = 802816
EPT = EPAD // 32
ESB = EPT // 4
CAP = 4800
BAT = 192
NB = CAP // BAT
NLIST = NCHUNK * 32 * CAP

f32 = jnp.float32
i32 = jnp.int32


def _mesh():
    return plsc.VectorSubcoreMesh(core_axis_name="c", subcore_axis_name="s")


def _sc_filter(src_p, dst_p):
    @functools.partial(
        pl.kernel,
        out_type=(
            jax.ShapeDtypeStruct((NLIST,), i32),
            jax.ShapeDtypeStruct((NLIST,), i32),
        ),
        mesh=_mesh(),
        compiler_params=pltpu.CompilerParams(needs_layout_passes=False),
        scratch_types=[
            pltpu.VMEM((NCHUNK * CAP,), i32),
            pltpu.VMEM((NCHUNK * CAP,), i32),
            pltpu.VMEM((ESB,), i32),
            pltpu.VMEM((ESB,), i32),
        ],
    )
    def k(src_hbm, dst_hbm, lsrc_hbm, ldst_hbm, bs, bd, sblk, dblk):
        cid = lax.axis_index("c")
        sid = lax.axis_index("s")
        tid = cid * 16 + sid
        iv = lax.iota(i32, 16)

        def pf(kk, _):
            sv = N + lax.rem(iv + kk, 176)
            dv = jnp.zeros((16,), i32) + CHUNK
            for c in range(NCHUNK):
                bs[pl.ds(c * CAP + kk * 16, 16)] = sv
                bd[pl.ds(c * CAP + kk * 16, 16)] = dv
            return 0

        lax.fori_loop(0, CAP // 16, pf, 0)

        base = pl.multiple_of(tid * EPT, 8)
        cnts = (jnp.zeros((), i32),) * NCHUNK
        for sb in range(4):
            pltpu.sync_copy(src_hbm.at[pl.ds(base + sb * ESB, ESB)], sblk)
            pltpu.sync_copy(dst_hbm.at[pl.ds(base + sb * ESB, ESB)], dblk)

            def fb(kk, cs):
                sl = pl.ds(kk * 16, 16)
                d = dblk[sl]
                s = sblk[sl]
                ch = lax.div(d, CHUNK)
                off = d - ch * CHUNK
                out = []
                for c in range(NCHUNK):
                    msk = ch == c
                    n = jnp.sum(msk.astype(i32))
                    cc = jnp.minimum(cs[c], CAP - 16)
                    plsc.store_compressed(bs.at[pl.ds(c * CAP + cc, 16)], s,
                                          mask=msk)
                    plsc.store_compressed(bd.at[pl.ds(c * CAP + cc, 16)], off,
                                          mask=msk)
                    out.append(cc + n)
                return tuple(out)

            cnts = lax.fori_loop(0, ESB // 16, fb, cnts)

        for c in range(NCHUNK):
            lb = pl.multiple_of((c * 32 + tid) * CAP, 8)
            pltpu.sync_copy(bs.at[pl.ds(c * CAP, CAP)],
                            lsrc_hbm.at[pl.ds(lb, CAP)])
            pltpu.sync_copy(bd.at[pl.ds(c * CAP, CAP)],
                            ldst_hbm.at[pl.ds(lb, CAP)])

    return k(src_p, dst_p)


def _sc_agg(table, lsrc, ldst, w):
    @functools.partial(
        pl.kernel,
        out_type=jax.ShapeDtypeStruct((NPAD, w), f32),
        mesh=_mesh(),
        compiler_params=pltpu.CompilerParams(needs_layout_passes=False,
                                             use_tc_tiling_on_sc=False),
        scratch_types=[
            pltpu.VMEM((CAP,), i32),
            pltpu.VMEM((CAP,), i32),
            pltpu.VMEM((BAT, w), f32),
            pltpu.VMEM((BAT, w), f32),
            pltpu.VMEM_SHARED((CHUNK + 16, w), f32),
            pltpu.SemaphoreType.DMA,
            pltpu.SemaphoreType.DMA,
        ],
    )
    def k(tab, lsrc_h, ldst_h, m_h, sbuf, dbuf, rows0, rows1, acc, sem0, sem1):
        cid = lax.axis_index("c")
        sid = lax.axis_index("s")
        z16 = jnp.zeros((16,), f32)
        sb0 = pl.multiple_of(sid * STRIPE, 16)
        rbufs = (rows0, rows1)
        sems = (sem0, sem1)
        for ci in range(3):
            c = 3 * cid + ci

            def zr(i, _):
                r = i // (w // 16)
                col = lax.rem(i, w // 16) * 16
                rows0[r, pl.ds(col, 16)] = z16
                return 0

            lax.fori_loop(0, BAT * (w // 16), zr, 0)
            off = 0
            while off < STRIPE:
                step = min(BAT, STRIPE - off)
                pltpu.sync_copy(rows0.at[pl.ds(0, step)],
                                acc.at[pl.ds(sb0 + off, step)])
                off += step
            plsc.subcore_barrier()

            for jj in range(2):
                j = 2 * sid + jj
                lb = pl.multiple_of((c * 32 + j) * CAP, 8)
                pltpu.sync_copy(lsrc_h.at[pl.ds(lb, CAP)], sbuf)
                pltpu.sync_copy(ldst_h.at[pl.ds(lb, CAP)], dbuf)

                hs = [None] * NB
                hs[0] = pltpu.async_copy(tab.at[sbuf.at[pl.ds(0, BAT)]],
                                         rows0, sem0)
                for b in range(NB):
                    hs[b].wait()
                    if b + 1 < NB:
                        hs[b + 1] = pltpu.async_copy(
                            tab.at[sbuf.at[pl.ds((b + 1) * BAT, BAT)]],
                            rbufs[(b + 1) % 2], sems[(b + 1) % 2])
                    pltpu.sync_copy(rbufs[b % 2],
                                    acc.at[dbuf.at[pl.ds(b * BAT, BAT)]],
                                    add=True)
            plsc.subcore_barrier()
            mb = c * CHUNK + sb0
            pltpu.sync_copy(acc.at[pl.ds(sb0, STRIPE)], m_h.at[pl.ds(mb, STRIPE)])
            plsc.subcore_barrier()

    return k(table, lsrc, ldst)


def _sc_deg(ldst):
    @functools.partial(
        pl.kernel,
        out_type=jax.ShapeDtypeStruct((NPAD, 16), f32),
        mesh=_mesh(),
        compiler_params=pltpu.CompilerParams(needs_layout_passes=False,
                                             use_tc_tiling_on_sc=False),
        scratch_types=[
            pltpu.VMEM((CAP,), i32),
            pltpu.VMEM((BAT, 16), f32),
            pltpu.VMEM_SHARED((CHUNK + 16, 16), f32),
        ],
    )
    def k(ldst_h, m_h, dbuf, rows, acc):
        cid = lax.axis_index("c")
        sid = lax.axis_index("s")
        z16 = jnp.zeros((16,), f32)
        sb0 = pl.multiple_of(sid * STRIPE, 16)
        for ci in range(3):
            c = 3 * cid + ci

            def zr(r, _):
                rows[r, pl.ds(0, 16)] = z16
                return 0

            lax.fori_loop(0, BAT, zr, 0)
            off = 0
            while off < STRIPE:
                step = min(BAT, STRIPE - off)
                pltpu.sync_copy(rows.at[pl.ds(0, step)],
                                acc.at[pl.ds(sb0 + off, step)])
                off += step
            plsc.subcore_barrier()

            ov = (lax.iota(i32, 16) == 0).astype(f32)

            def fr(r, _):
                rows[r, pl.ds(0, 16)] = ov
                return 0

            lax.fori_loop(0, BAT, fr, 0)
            for jj in range(2):
                j = 2 * sid + jj
                lb = pl.multiple_of((c * 32 + j) * CAP, 8)
                pltpu.sync_copy(ldst_h.at[pl.ds(lb, CAP)], dbuf)

                def bat(b, _):
                    pltpu.sync_copy(rows,
                                    acc.at[dbuf.at[pl.ds(b * BAT, BAT)]],
                                    add=True)
                    return 0

                lax.fori_loop(0, NB, bat, 0)
            plsc.subcore_barrier()
            mb = c * CHUNK + sb0
            pltpu.sync_copy(acc.at[pl.ds(sb0, STRIPE)], m_h.at[pl.ds(mb, STRIPE)])
            plsc.subcore_barrier()

    return k(ldst)


def _rowid(i):
    return lax.broadcasted_iota(i32, (BLK, 1), 0) + i * BLK


def _tc_prep(m1, xpad):
    def body(dp_ref, x_ref, dinv_ref, gx_ref):
        i = pl.program_id(0)
        deg = dp_ref[:, 0:1] + 1.0
        dinv = jnp.where(_rowid(i) < N,
                         lax.rsqrt(jnp.maximum(deg, 1.0)), 0.0)
        dinv_ref[...] = dinv
        gx_ref[...] = x_ref[...] * dinv

    return pl.pallas_call(
        body,
        grid=(NBLK,),
        in_specs=[
            pl.BlockSpec((BLK, 16), lambda i: (i, 0)),
            pl.BlockSpec((BLK, 16), lambda i: (i, 0)),
        ],
        out_specs=[
            pl.BlockSpec((BLK, 1), lambda i: (i, 0)),
            pl.BlockSpec((BLK, 16), lambda i: (i, 0)),
        ],
        out_shape=[
            jax.ShapeDtypeStruct((NPAD, 1), f32),
            jax.ShapeDtypeStruct((NPAD, 16), f32),
        ],
    )(m1, xpad)


def _stats_update(i, y, scr, st_ref):
    ym = jnp.where(_rowid(i) < N, y, 0.0)

    @pl.when(i == 0)
    def _():
        scr[...] = jnp.zeros_like(scr)

    scr[0:1, :] += jnp.sum(ym, axis=0, keepdims=True)
    scr[1:2, :] += jnp.sum(ym * ym, axis=0, keepdims=True)
    st_ref[...] = scr[...]


def _tc_post1(mx, gx, dinv, w1p, b1r):
    def body(m_ref, g_ref, dv_ref, w_ref, b_ref, y_ref, st_ref, scr):
        i = pl.program_id(0)
        t = dv_ref[...] * (m_ref[...] + g_ref[...])
        y = jnp.dot(t, w_ref[...], preferred_element_type=f32) + b_ref[...]
        y_ref[...] = y
        _stats_update(i, y, scr, st_ref)

    return pl.pallas_call(
        body,
        grid=(NBLK,),
        in_specs=[
            pl.BlockSpec((BLK, 16), lambda i: (i, 0)),
            pl.BlockSpec((BLK, 16), lambda i: (i, 0)),
            pl.BlockSpec((BLK, 1), lambda i: (i, 0)),
            pl.BlockSpec((16, H), lambda i: (0, 0)),
            pl.BlockSpec((1, H), lambda i: (0, 0)),
        ],
        out_specs=[
            pl.BlockSpec((BLK, H), lambda i: (i, 0)),
            pl.BlockSpec((2, H), lambda i: (0, 0)),
        ],
        out_shape=[
            jax.ShapeDtypeStruct((NPAD, H), f32),
            jax.ShapeDtypeStruct((2, H), f32),
        ],
        scratch_shapes=[pltpu.VMEM((2, H), f32)],
    )(mx, gx, dinv, w1p, b1r)


def _tc_post23(m, g, dinv, br):
    def body(m_ref, g_ref, dv_ref, b_ref, y_ref, st_ref, scr):
        i = pl.program_id(0)
        y = dv_ref[...] * (m_ref[...] + g_ref[...]) + b_ref[...]
        y_ref[...] = y
        _stats_update(i, y, scr, st_ref)

    return pl.pallas_call(
        body,
        grid=(NBLK,),
        in_specs=[
            pl.BlockSpec((BLK, H), lambda i: (i, 0)),
            pl.BlockSpec((BLK, H), lambda i: (i, 0)),
            pl.BlockSpec((BLK, 1), lambda i: (i, 0)),
            pl.BlockSpec((1, H), lambda i: (0, 0)),
        ],
        out_specs=[
            pl.BlockSpec((BLK, H), lambda i: (i, 0)),
            pl.BlockSpec((2, H), lambda i: (0, 0)),
        ],
        out_shape=[
            jax.ShapeDtypeStruct((NPAD, H), f32),
            jax.ShapeDtypeStruct((2, H), f32),
        ],
        scratch_shapes=[pltpu.VMEM((2, H), f32)],
    )(m, g, dinv, br)


def _bn_relu(y, st_ref, ga_ref, be_ref):
    mu = st_ref[0:1, :] * (1.0 / N)
    var = st_ref[1:2, :] * (1.0 / N) - mu * mu
    rstd = lax.rsqrt(var + EPS)
    return jnp.maximum(ga_ref[...] * (y - mu) * rstd + be_ref[...], 0.0)


def _tc_mid(y, st, gar, ber, dinv, wn):
    def body(y_ref, st_ref, ga_ref, be_ref, dv_ref, w_ref, gn_ref):
        h = _bn_relu(y_ref[...], st_ref, ga_ref, be_ref)
        gn_ref[...] = dv_ref[...] * jnp.dot(h, w_ref[...],
                                            preferred_element_type=f32)

    return pl.pallas_call(
        body,
        grid=(NBLK,),
        in_specs=[
            pl.BlockSpec((BLK, H), lambda i: (i, 0)),
            pl.BlockSpec((2, H), lambda i: (0, 0)),
            pl.BlockSpec((1, H), lambda i: (0, 0)),
            pl.BlockSpec((1, H), lambda i: (0, 0)),
            pl.BlockSpec((BLK, 1), lambda i: (i, 0)),
            pl.BlockSpec((H, H), lambda i: (0, 0)),
        ],
        out_specs=pl.BlockSpec((BLK, H), lambda i: (i, 0)),
        out_shape=jax.ShapeDtypeStruct((NPAD, H), f32),
    )(y, st, gar, ber, dinv, wn)


def _tc_final(y3, s3, g3r, be3r, bcol, fw1, fb1r, fw2, fb2r, fw3p, fb3p):
    def body(y_ref, st_ref, ga_ref, be_ref, b_ref, w1_ref, b1_ref, w2_ref,
             b2_ref, w3_ref, b3_ref, out_ref, pool, cnt):
        i = pl.program_id(0)
        h = _bn_relu(y_ref[...], st_ref, ga_ref, be_ref)
        oh = (b_ref[...] == lax.broadcasted_iota(i32, (1, G), 1)).astype(f32)

        @pl.when(i == 0)
        def _():
            pool[...] = jnp.zeros_like(pool)
            cnt[...] = jnp.zeros_like(cnt)

        dn = (((0,), (0,)), ((), ()))
        pool[...] += lax.dot_general(oh, h, dn, preferred_element_type=f32)
        cnt[...] += lax.dot_general(oh, jnp.ones((BLK, 1), f32), dn,
                                    preferred_element_type=f32)

        @pl.when(i == NBLK - 1)
        def _():
            pm = pool[...] / jnp.maximum(cnt[...], 1.0)
            z = jnp.maximum(
                jnp.dot(pm, w1_ref[...], preferred_element_type=f32)
                + b1_ref[...], 0.0)
            z = jnp.maximum(
                jnp.dot(z, w2_ref[...], preferred_element_type=f32)
                + b2_ref[...], 0.0)
            out_ref[...] = (jnp.dot(z, w3_ref[...], preferred_element_type=f32)
                            + b3_ref[...])

    return pl.pallas_call(
        body,
        grid=(NBLK,),
        in_specs=[
            pl.BlockSpec((BLK, H), lambda i: (i, 0)),
            pl.BlockSpec((2, H), lambda i: (0, 0)),
            pl.BlockSpec((1, H), lambda i: (0, 0)),
            pl.BlockSpec((1, H), lambda i: (0, 0)),
            pl.BlockSpec((BLK, 1), lambda i: (i, 0)),
            pl.BlockSpec((H, L), lambda i: (0, 0)),
            pl.BlockSpec((1, L), lambda i: (0, 0)),
            pl.BlockSpec((L, L), lambda i: (0, 0)),
            pl.BlockSpec((1, L), lambda i: (0, 0)),
            pl.BlockSpec((L, H), lambda i: (0, 0)),
            pl.BlockSpec((1, H), lambda i: (0, 0)),
        ],
        out_specs=pl.BlockSpec((G, H), lambda i: (0, 0)),
        out_shape=jax.ShapeDtypeStruct((G, H), f32),
        scratch_shapes=[pltpu.VMEM((G, H), f32), pltpu.VMEM((G, 1), f32)],
    )(y3, s3, g3r, be3r, bcol, fw1, fb1r, fw2, fb2r, fw3p, fb3p)


def kernel(x, edge_index, batch, W1, b1, g1, be1, W2, b2, g2, be2,
           W3, b3, g3, be3, fW1, fb1, fW2, fb2, fW3, fb3):
    extra = EPAD - E
    ar = jnp.arange(extra, dtype=i32)
    src_p = jnp.concatenate([edge_index[0], ar % 512])
    dst_p = jnp.concatenate([edge_index[1], N + ar % 176])
    xpad = jnp.pad(x, ((0, NPAD - N), (0, 16 - F)))
    bcol = jnp.pad(batch, (0, NPAD - N), constant_values=G).reshape(NPAD, 1)
    w1p = jnp.pad(W1, ((0, 16 - F), (0, 0)))
    fw3p = jnp.pad(fW3, ((0, 0), (0, H - 1)))
    fb3p = jnp.pad(fb3, (0, H - 1)).reshape(1, H)

    lsrc, ldst = _sc_filter(src_p, dst_p)
    m1 = _sc_deg(ldst)
    dinv, gx = _tc_prep(m1, xpad)

    mx = _sc_agg(gx, lsrc, ldst, 16)
    y1, s1 = _tc_post1(mx, gx, dinv, w1p, b1.reshape(1, H))
    g2_ = _tc_mid(y1, s1, g1.reshape(1, H), be1.reshape(1, H), dinv, W2)

    m2 = _sc_agg(g2_, lsrc, ldst, H)
    y2, s2 = _tc_post23(m2, g2_, dinv, b2.reshape(1, H))
    g3_ = _tc_mid(y2, s2, g2.reshape(1, H), be2.reshape(1, H), dinv, W3)

    m3 = _sc_agg(g3_, lsrc, ldst, H)
    y3, s3 = _tc_post23(m3, g3_, dinv, b3.reshape(1, H))

    out2d = _tc_final(y3, s3, g3.reshape(1, H), be3.reshape(1, H), bcol,
                      fW1, fb1.reshape(1, L), fW2, fb2.reshape(1, L),
                      fw3p, fb3p)
    return out2d[:, 0]


# speedup vs baseline: 21.4577x; 1.4220x over previous
import functools

import jax
import jax.numpy as jnp
from jax import lax
from jax.experimental import pallas as pl
from jax.experimental.pallas import tpu as pltpu
from jax.experimental.pallas import tpu_sc as plsc

N = 50000
E = 800000
F = 10
H = 128
L = 256
G = 512
EPS = 1e-5

NPAD = 50688
BLK = 512
NBLK = NPAD // BLK
CHUNK = 8448
NCHUNK = 6
STRIPE = CHUNK // 16
EPAD = 802816
EPT = EPAD // 32
ESB = EPT // 4
CAP = 4800
BAT = 192
NB = CAP // BAT
NLIST = NCHUNK * 32 * CAP

f32 = jnp.float32
i32 = jnp.int32


def _mesh():
    return plsc.VectorSubcoreMesh(core_axis_name="c", subcore_axis_name="s")


def _sc_filter(src_p, dst_p):
    @functools.partial(
        pl.kernel,
        out_type=(
            jax.ShapeDtypeStruct((NLIST,), i32),
            jax.ShapeDtypeStruct((NLIST,), i32),
            jax.ShapeDtypeStruct((32 * 16,), i32),
        ),
        mesh=_mesh(),
        compiler_params=pltpu.CompilerParams(needs_layout_passes=False),
        scratch_types=[
            pltpu.VMEM((NCHUNK * CAP,), i32),
            pltpu.VMEM((NCHUNK * CAP,), i32),
            pltpu.VMEM((ESB,), i32),
            pltpu.VMEM((ESB,), i32),
            pltpu.VMEM((16,), i32),
        ],
    )
    def k(src_hbm, dst_hbm, lsrc_hbm, ldst_hbm, cnt_hbm, bs, bd, sblk, dblk,
          cbuf):
        cid = lax.axis_index("c")
        sid = lax.axis_index("s")
        tid = cid * 16 + sid
        iv = lax.iota(i32, 16)

        def pf(kk, _):
            sv = N + lax.rem(iv + kk, 176)
            dv = jnp.zeros((16,), i32) + CHUNK
            for c in range(NCHUNK):
                bs[pl.ds(c * CAP + kk * 16, 16)] = sv
                bd[pl.ds(c * CAP + kk * 16, 16)] = dv
            return 0

        lax.fori_loop(0, CAP // 16, pf, 0)

        base = pl.multiple_of(tid * EPT, 8)
        cnts = (jnp.zeros((), i32),) * NCHUNK
        for sb in range(4):
            pltpu.sync_copy(src_hbm.at[pl.ds(base + sb * ESB, ESB)], sblk)
            pltpu.sync_copy(dst_hbm.at[pl.ds(base + sb * ESB, ESB)], dblk)

            def fb(kk, cs):
                sl = pl.ds(kk * 16, 16)
                d = dblk[sl]
                s = sblk[sl]
                ch = lax.div(d, CHUNK)
                off = d - ch * CHUNK
                out = []
                for c in range(NCHUNK):
                    msk = ch == c
                    n = jnp.sum(msk.astype(i32))
                    cc = jnp.minimum(cs[c], CAP - 16)
                    plsc.store_compressed(bs.at[pl.ds(c * CAP + cc, 16)], s,
                                          mask=msk)
                    plsc.store_compressed(bd.at[pl.ds(c * CAP + cc, 16)], off,
                                          mask=msk)
                    out.append(cc + n)
                return tuple(out)

            cnts = lax.fori_loop(0, ESB // 16, fb, cnts)

        for c in range(NCHUNK):
            lb = pl.multiple_of((c * 32 + tid) * CAP, 8)
            pltpu.sync_copy(bs.at[pl.ds(c * CAP, CAP)],
                            lsrc_hbm.at[pl.ds(lb, CAP)])
            pltpu.sync_copy(bd.at[pl.ds(c * CAP, CAP)],
                            ldst_hbm.at[pl.ds(lb, CAP)])

        cv = jnp.zeros((16,), i32)
        for c in range(NCHUNK):
            cv = jnp.where(iv == c, cnts[c], cv)
        cbuf[pl.ds(0, 16)] = cv
        cb = pl.multiple_of(tid * 16, 8)
        pltpu.sync_copy(cbuf, cnt_hbm.at[pl.ds(cb, 16)])

    return k(src_p, dst_p)


def _sc_agg(table, lsrc, ldst, lcnt, w):
    @functools.partial(
        pl.kernel,
        out_type=jax.ShapeDtypeStruct((NPAD, w), f32),
        mesh=_mesh(),
        compiler_params=pltpu.CompilerParams(needs_layout_passes=False,
                                             use_tc_tiling_on_sc=False),
        scratch_types=[
            pltpu.VMEM((CAP,), i32),
            pltpu.VMEM((CAP,), i32),
            pltpu.VMEM((32,), i32),
            pltpu.VMEM((BAT, w), f32),
            pltpu.VMEM((BAT, w), f32),
            pltpu.VMEM_SHARED((CHUNK + 16, w), f32),
            pltpu.SemaphoreType.DMA,
            pltpu.SemaphoreType.DMA,
        ],
    )
    def k(tab, lsrc_h, ldst_h, lcnt_h, m_h, sbuf, dbuf, ccb, rows0, rows1,
          acc, sem0, sem1):
        cid = lax.axis_index("c")
        sid = lax.axis_index("s")
        iv = lax.iota(i32, 16)
        z16 = jnp.zeros((16,), f32)
        sb0 = pl.multiple_of(sid * STRIPE, 16)
        rbufs = (rows0, rows1)
        sems = (sem0, sem1)
        lc = pl.multiple_of(sid * 32, 8)
        pltpu.sync_copy(lcnt_h.at[pl.ds(lc, 32)], ccb)
        cvs = (ccb[pl.ds(0, 16)], ccb[pl.ds(16, 16)])
        for ci in range(3):
            c = 3 * cid + ci

            def zr(i, _):
                r = i // (w // 16)
                col = lax.rem(i, w // 16) * 16
                rows0[r, pl.ds(col, 16)] = z16
                return 0

            lax.fori_loop(0, BAT * (w // 16), zr, 0)
            off = 0
            while off < STRIPE:
                step = min(BAT, STRIPE - off)
                pltpu.sync_copy(rows0.at[pl.ds(0, step)],
                                acc.at[pl.ds(sb0 + off, step)])
                off += step
            plsc.subcore_barrier()

            for jj in range(2):
                j = 2 * sid + jj
                lb = pl.multiple_of((c * 32 + j) * CAP, 8)
                pltpu.sync_copy(lsrc_h.at[pl.ds(lb, CAP)], sbuf)
                pltpu.sync_copy(ldst_h.at[pl.ds(lb, CAP)], dbuf)
                cnt = jnp.sum(jnp.where(iv == c, cvs[jj], 0))

                def issue(b):
                    pltpu.async_copy(tab.at[sbuf.at[pl.ds(b * BAT, BAT)]],
                                     rbufs[b % 2], sems[b % 2])

                def drain_scat(b):
                    pltpu.make_async_copy(m_h.at[pl.ds(0, BAT)],
                                          rbufs[b % 2], sems[b % 2]).wait()
                    pltpu.sync_copy(rbufs[b % 2],
                                    acc.at[dbuf.at[pl.ds(b * BAT, BAT)]],
                                    add=True)

                issue(0)
                for b in range(NB):
                    if b + 1 < NB:
                        @pl.when((b + 1) * BAT < cnt)
                        def _(b=b):
                            issue(b + 1)
                    if b == 0:
                        drain_scat(0)
                    else:
                        @pl.when(b * BAT < cnt)
                        def _(b=b):
                            drain_scat(b)
            plsc.subcore_barrier()
            mb = c * CHUNK + sb0
            pltpu.sync_copy(acc.at[pl.ds(sb0, STRIPE)], m_h.at[pl.ds(mb, STRIPE)])
            plsc.subcore_barrier()

    return k(table, lsrc, ldst, lcnt)


def _sc_deg(ldst, lcnt):
    @functools.partial(
        pl.kernel,
        out_type=jax.ShapeDtypeStruct((NPAD, 16), f32),
        mesh=_mesh(),
        compiler_params=pltpu.CompilerParams(needs_layout_passes=False,
                                             use_tc_tiling_on_sc=False),
        scratch_types=[
            pltpu.VMEM((CAP,), i32),
            pltpu.VMEM((32,), i32),
            pltpu.VMEM((BAT, 16), f32),
            pltpu.VMEM_SHARED((CHUNK + 16, 16), f32),
        ],
    )
    def k(ldst_h, lcnt_h, m_h, dbuf, ccb, rows, acc):
        cid = lax.axis_index("c")
        sid = lax.axis_index("s")
        iv = lax.iota(i32, 16)
        z16 = jnp.zeros((16,), f32)
        sb0 = pl.multiple_of(sid * STRIPE, 16)
        lc = pl.multiple_of(sid * 32, 8)
        pltpu.sync_copy(lcnt_h.at[pl.ds(lc, 32)], ccb)
        cvs = (ccb[pl.ds(0, 16)], ccb[pl.ds(16, 16)])
        for ci in range(3):
            c = 3 * cid + ci

            def zr(r, _):
                rows[r, pl.ds(0, 16)] = z16
                return 0

            lax.fori_loop(0, BAT, zr, 0)
            off = 0
            while off < STRIPE:
                step = min(BAT, STRIPE - off)
                pltpu.sync_copy(rows.at[pl.ds(0, step)],
                                acc.at[pl.ds(sb0 + off, step)])
                off += step
            plsc.subcore_barrier()

            ov = (lax.iota(i32, 16) == 0).astype(f32)

            def fr(r, _):
                rows[r, pl.ds(0, 16)] = ov
                return 0

            lax.fori_loop(0, BAT, fr, 0)
            for jj in range(2):
                j = 2 * sid + jj
                lb = pl.multiple_of((c * 32 + j) * CAP, 8)
                pltpu.sync_copy(ldst_h.at[pl.ds(lb, CAP)], dbuf)
                cnt = jnp.sum(jnp.where(iv == c, cvs[jj], 0))
                nbd = jnp.maximum(lax.div(cnt + (BAT - 1), BAT), 1)

                def bat(b, _):
                    pltpu.sync_copy(rows,
                                    acc.at[dbuf.at[pl.ds(b * BAT, BAT)]],
                                    add=True)
                    return 0

                lax.fori_loop(0, nbd, bat, 0)
            plsc.subcore_barrier()
            mb = c * CHUNK + sb0
            pltpu.sync_copy(acc.at[pl.ds(sb0, STRIPE)], m_h.at[pl.ds(mb, STRIPE)])
            plsc.subcore_barrier()

    return k(ldst, lcnt)


def _rowid(i):
    return lax.broadcasted_iota(i32, (BLK, 1), 0) + i * BLK


def _tc_prep(m1, xpad):
    def body(dp_ref, x_ref, dinv_ref, gx_ref):
        i = pl.program_id(0)
        deg = dp_ref[:, 0:1] + 1.0
        dinv = jnp.where(_rowid(i) < N,
                         lax.rsqrt(jnp.maximum(deg, 1.0)), 0.0)
        dinv_ref[...] = dinv
        gx_ref[...] = x_ref[...] * dinv

    return pl.pallas_call(
        body,
        grid=(NBLK,),
        in_specs=[
            pl.BlockSpec((BLK, 16), lambda i: (i, 0)),
            pl.BlockSpec((BLK, 16), lambda i: (i, 0)),
        ],
        out_specs=[
            pl.BlockSpec((BLK, 1), lambda i: (i, 0)),
            pl.BlockSpec((BLK, 16), lambda i: (i, 0)),
        ],
        out_shape=[
            jax.ShapeDtypeStruct((NPAD, 1), f32),
            jax.ShapeDtypeStruct((NPAD, 16), f32),
        ],
    )(m1, xpad)


def _stats_update(i, y, scr, st_ref):
    ym = jnp.where(_rowid(i) < N, y, 0.0)

    @pl.when(i == 0)
    def _():
        scr[...] = jnp.zeros_like(scr)

    scr[0:1, :] += jnp.sum(ym, axis=0, keepdims=True)
    scr[1:2, :] += jnp.sum(ym * ym, axis=0, keepdims=True)
    st_ref[...] = scr[...]


def _tc_post1(mx, gx, dinv, w1p, b1r):
    def body(m_ref, g_ref, dv_ref, w_ref, b_ref, y_ref, st_ref, scr):
        i = pl.program_id(0)
        t = dv_ref[...] * (m_ref[...] + g_ref[...])
        y = jnp.dot(t, w_ref[...], preferred_element_type=f32) + b_ref[...]
        y_ref[...] = y
        _stats_update(i, y, scr, st_ref)

    return pl.pallas_call(
        body,
        grid=(NBLK,),
        in_specs=[
            pl.BlockSpec((BLK, 16), lambda i: (i, 0)),
            pl.BlockSpec((BLK, 16), lambda i: (i, 0)),
            pl.BlockSpec((BLK, 1), lambda i: (i, 0)),
            pl.BlockSpec((16, H), lambda i: (0, 0)),
            pl.BlockSpec((1, H), lambda i: (0, 0)),
        ],
        out_specs=[
            pl.BlockSpec((BLK, H), lambda i: (i, 0)),
            pl.BlockSpec((2, H), lambda i: (0, 0)),
        ],
        out_shape=[
            jax.ShapeDtypeStruct((NPAD, H), f32),
            jax.ShapeDtypeStruct((2, H), f32),
        ],
        scratch_shapes=[pltpu.VMEM((2, H), f32)],
    )(mx, gx, dinv, w1p, b1r)


def _tc_post23(m, g, dinv, br):
    def body(m_ref, g_ref, dv_ref, b_ref, y_ref, st_ref, scr):
        i = pl.program_id(0)
        y = dv_ref[...] * (m_ref[...] + g_ref[...]) + b_ref[...]
        y_ref[...] = y
        _stats_update(i, y, scr, st_ref)

    return pl.pallas_call(
        body,
        grid=(NBLK,),
        in_specs=[
            pl.BlockSpec((BLK, H), lambda i: (i, 0)),
            pl.BlockSpec((BLK, H), lambda i: (i, 0)),
            pl.BlockSpec((BLK, 1), lambda i: (i, 0)),
            pl.BlockSpec((1, H), lambda i: (0, 0)),
        ],
        out_specs=[
            pl.BlockSpec((BLK, H), lambda i: (i, 0)),
            pl.BlockSpec((2, H), lambda i: (0, 0)),
        ],
        out_shape=[
            jax.ShapeDtypeStruct((NPAD, H), f32),
            jax.ShapeDtypeStruct((2, H), f32),
        ],
        scratch_shapes=[pltpu.VMEM((2, H), f32)],
    )(m, g, dinv, br)


def _bn_relu(y, st_ref, ga_ref, be_ref):
    mu = st_ref[0:1, :] * (1.0 / N)
    var = st_ref[1:2, :] * (1.0 / N) - mu * mu
    rstd = lax.rsqrt(var + EPS)
    return jnp.maximum(ga_ref[...] * (y - mu) * rstd + be_ref[...], 0.0)


def _tc_mid(y, st, gar, ber, dinv, wn):
    def body(y_ref, st_ref, ga_ref, be_ref, dv_ref, w_ref, gn_ref):
        h = _bn_relu(y_ref[...], st_ref, ga_ref, be_ref)
        gn_ref[...] = dv_ref[...] * jnp.dot(h, w_ref[...],
                                            preferred_element_type=f32)

    return pl.pallas_call(
        body,
        grid=(NBLK,),
        in_specs=[
            pl.BlockSpec((BLK, H), lambda i: (i, 0)),
            pl.BlockSpec((2, H), lambda i: (0, 0)),
            pl.BlockSpec((1, H), lambda i: (0, 0)),
            pl.BlockSpec((1, H), lambda i: (0, 0)),
            pl.BlockSpec((BLK, 1), lambda i: (i, 0)),
            pl.BlockSpec((H, H), lambda i: (0, 0)),
        ],
        out_specs=pl.BlockSpec((BLK, H), lambda i: (i, 0)),
        out_shape=jax.ShapeDtypeStruct((NPAD, H), f32),
    )(y, st, gar, ber, dinv, wn)


def _tc_final(y3, s3, g3r, be3r, bcol, fw1, fb1r, fw2, fb2r, fw3p, fb3p):
    def body(y_ref, st_ref, ga_ref, be_ref, b_ref, w1_ref, b1_ref, w2_ref,
             b2_ref, w3_ref, b3_ref, out_ref, pool, cnt):
        i = pl.program_id(0)
        h = _bn_relu(y_ref[...], st_ref, ga_ref, be_ref)
        oh = (b_ref[...] == lax.broadcasted_iota(i32, (1, G), 1)).astype(f32)

        @pl.when(i == 0)
        def _():
            pool[...] = jnp.zeros_like(pool)
            cnt[...] = jnp.zeros_like(cnt)

        dn = (((0,), (0,)), ((), ()))
        pool[...] += lax.dot_general(oh, h, dn, preferred_element_type=f32)
        cnt[...] += lax.dot_general(oh, jnp.ones((BLK, 1), f32), dn,
                                    preferred_element_type=f32)

        @pl.when(i == NBLK - 1)
        def _():
            pm = pool[...] / jnp.maximum(cnt[...], 1.0)
            z = jnp.maximum(
                jnp.dot(pm, w1_ref[...], preferred_element_type=f32)
                + b1_ref[...], 0.0)
            z = jnp.maximum(
                jnp.dot(z, w2_ref[...], preferred_element_type=f32)
                + b2_ref[...], 0.0)
            out_ref[...] = (jnp.dot(z, w3_ref[...], preferred_element_type=f32)
                            + b3_ref[...])

    return pl.pallas_call(
        body,
        grid=(NBLK,),
        in_specs=[
            pl.BlockSpec((BLK, H), lambda i: (i, 0)),
            pl.BlockSpec((2, H), lambda i: (0, 0)),
            pl.BlockSpec((1, H), lambda i: (0, 0)),
            pl.BlockSpec((1, H), lambda i: (0, 0)),
            pl.BlockSpec((BLK, 1), lambda i: (i, 0)),
            pl.BlockSpec((H, L), lambda i: (0, 0)),
            pl.BlockSpec((1, L), lambda i: (0, 0)),
            pl.BlockSpec((L, L), lambda i: (0, 0)),
            pl.BlockSpec((1, L), lambda i: (0, 0)),
            pl.BlockSpec((L, H), lambda i: (0, 0)),
            pl.BlockSpec((1, H), lambda i: (0, 0)),
        ],
        out_specs=pl.BlockSpec((G, H), lambda i: (0, 0)),
        out_shape=jax.ShapeDtypeStruct((G, H), f32),
        scratch_shapes=[pltpu.VMEM((G, H), f32), pltpu.VMEM((G, 1), f32)],
    )(y3, s3, g3r, be3r, bcol, fw1, fb1r, fw2, fb2r, fw3p, fb3p)


def kernel(x, edge_index, batch, W1, b1, g1, be1, W2, b2, g2, be2,
           W3, b3, g3, be3, fW1, fb1, fW2, fb2, fW3, fb3):
    extra = EPAD - E
    ar = jnp.arange(extra, dtype=i32)
    src_p = jnp.concatenate([edge_index[0], ar % 512])
    dst_p = jnp.concatenate([edge_index[1], N + ar % 176])
    xpad = jnp.pad(x, ((0, NPAD - N), (0, 16 - F)))
    bcol = jnp.pad(batch, (0, NPAD - N), constant_values=G).reshape(NPAD, 1)
    w1p = jnp.pad(W1, ((0, 16 - F), (0, 0)))
    fw3p = jnp.pad(fW3, ((0, 0), (0, H - 1)))
    fb3p = jnp.pad(fb3, (0, H - 1)).reshape(1, H)

    lsrc, ldst, lcnt = _sc_filter(src_p, dst_p)
    m1 = _sc_deg(ldst, lcnt)
    dinv, gx = _tc_prep(m1, xpad)

    mx = _sc_agg(gx, lsrc, ldst, lcnt, 16)
    y1, s1 = _tc_post1(mx, gx, dinv, w1p, b1.reshape(1, H))
    g2_ = _tc_mid(y1, s1, g1.reshape(1, H), be1.reshape(1, H), dinv, W2)

    m2 = _sc_agg(g2_, lsrc, ldst, lcnt, H)
    y2, s2 = _tc_post23(m2, g2_, dinv, b2.reshape(1, H))
    g3_ = _tc_mid(y2, s2, g2.reshape(1, H), be2.reshape(1, H), dinv, W3)

    m3 = _sc_agg(g3_, lsrc, ldst, lcnt, H)
    y3, s3 = _tc_post23(m3, g3_, dinv, b3.reshape(1, H))

    out2d = _tc_final(y3, s3, g3.reshape(1, H), be3.reshape(1, H), bcol,
                      fW1, fb1.reshape(1, L), fW2, fb2.reshape(1, L),
                      fw3p, fb3p)
    return out2d[:, 0]
